# baseline (speedup 1.0000x reference)
import jax
import jax.numpy as jnp
import numpy as np
from jax.experimental import pallas as pl

_B = 16
_N = 1024
_D = 1152
_K = 16
_OUT = 32
_BO = _B * _OUT
_GB = 4
_NP = _N // 8


def _silu(v):
    return v * jax.nn.sigmoid(v)


def _cond_kernel(t_ref, fr_ref, wt1_ref, bt1_ref, wt2_ref, bt2_ref,
                 y_ref, ytab_ref, s_ref):
    args = t_ref[...] * fr_ref[...]
    emb = jnp.concatenate([jnp.sin(args), jnp.cos(args)], axis=-1)
    h = jnp.dot(emb, wt1_ref[...], preferred_element_type=jnp.float32)
    h = _silu(h + bt1_ref[...])
    temb = jnp.dot(h, wt2_ref[...], preferred_element_type=jnp.float32)
    temb = temb + bt2_ref[...]
    n_cls = ytab_ref.shape[0]
    iota = jax.lax.broadcasted_iota(jnp.int32, (_B, n_cls), 1)
    onehot = (iota == y_ref[...]).astype(jnp.float32)
    yemb = jnp.dot(onehot, ytab_ref[...], preferred_element_type=jnp.float32)
    s_ref[...] = _silu(temb + yemb)


def _pre_kernel(s_ref, wada_ref, bada_ref, pos_ref, bp_ref, wpt_ref,
                wproj_ref, bproj_ref,
                qw_ref, pb_ref, pn_ref, gw8_ref, s16_ref, w8bar_ref,
                mball8_ref, posw_ref, soff8_ref):
    ada = jnp.dot(s_ref[...], wada_ref[...], preferred_element_type=jnp.float32)
    ada = ada + bada_ref[...]
    shift = ada[:, :_D]
    sc1 = 1.0 + ada[:, _D:]

    posq = pos_ref[...] + bp_ref[...]
    pbar = jnp.mean(posq, axis=1, keepdims=True)
    pnorm = jnp.sum(posq * posq, axis=1, keepdims=True)
    qw = jnp.dot(posq, wpt_ref[...], preferred_element_type=jnp.float32)

    wpt = wpt_ref[...]
    g = jax.lax.dot_general(wpt, wpt, (((0,), (0,)), ((), ())),
                            preferred_element_type=jnp.float32)
    wbarc = jax.lax.dot_general(
        wpt, jnp.full((_D, 1), 1.0 / _D, jnp.float32),
        (((0,), (0,)), ((), ())), preferred_element_type=jnp.float32)

    bi = jax.lax.broadcasted_iota(jnp.int32, (_B, _BO), 0)
    ci = jax.lax.broadcasted_iota(jnp.int32, (_B, _BO), 1)
    rsel = (bi == ci // _OUT).astype(jnp.float32)
    oi = jax.lax.broadcasted_iota(jnp.int32, (_OUT, _BO), 0)
    cj = jax.lax.broadcasted_iota(jnp.int32, (_OUT, _BO), 1)
    tsel = (oi == cj % _OUT).astype(jnp.float32)
    scale_exp = jax.lax.dot_general(sc1, rsel, (((0,), (0,)), ((), ())),
                                    preferred_element_type=jnp.float32)
    wtile = jnp.dot(wproj_ref[...], tsel, preferred_element_type=jnp.float32)
    wball = (scale_exp * wtile).astype(jnp.bfloat16)

    posw = jnp.dot(posq.astype(jnp.bfloat16), wball,
                   preferred_element_type=jnp.float32)
    mball = jax.lax.dot_general(
        wpt.astype(jnp.bfloat16), wball, (((0,), (0,)), ((), ())),
        preferred_element_type=jnp.float32)

    s_all = jnp.dot(sc1, wproj_ref[...], preferred_element_type=jnp.float32)
    off_all = jnp.dot(shift, wproj_ref[...],
                      preferred_element_type=jnp.float32) + bproj_ref[...]

    qw_ref[...] = qw
    pb_ref[...] = pbar
    pn_ref[...] = pnorm

    r128 = jax.lax.broadcasted_iota(jnp.int32, (8 * _K, 8 * _K), 0)
    c128 = jax.lax.broadcasted_iota(jnp.int32, (8 * _K, 8 * _K), 1)
    gw8_ref[...] = jnp.tile(g, (8, 8)) * (r128 // _K == c128 // _K
                                          ).astype(jnp.float32)
    r16 = jax.lax.broadcasted_iota(jnp.int32, (8 * _K, 8), 0)
    c16 = jax.lax.broadcasted_iota(jnp.int32, (8 * _K, 8), 1)
    diag8 = (r16 // _K == c16).astype(jnp.float32)
    s16_ref[...] = diag8
    w8bar_ref[...] = diag8 * jnp.tile(wbarc, (8, 1))

    rq = jax.lax.broadcasted_iota(jnp.int32, (8 * _K, 8 * _OUT), 0)
    cq = jax.lax.broadcasted_iota(jnp.int32, (8 * _K, 8 * _OUT), 1)
    mmask = (rq // _K == cq // _OUT).astype(jnp.float32)
    posw_ref[...] = posw
    for b in range(_B):
        mb = mball[:, b * _OUT:(b + 1) * _OUT]
        mball8_ref[b] = (jnp.tile(mb, (8, 8)) * mmask).astype(jnp.bfloat16)
        soff8_ref[b] = jnp.concatenate(
            [jnp.tile(s_all[b:b + 1], (1, 8)),
             jnp.tile(off_all[b:b + 1], (1, 8))], axis=0)


def _main_kernel(xt_ref, gw8_ref, s16_ref, w8bar_ref, qw8_ref, pb8_ref,
                 pn8_ref, mball8_ref, posw8_ref, soff8_ref, out_ref):
    gw8 = gw8_ref[...]
    s16 = s16_ref[...]
    w8bar = w8bar_ref[...]
    qw8 = qw8_ref[...]
    pb8 = pb8_ref[...]
    pn8 = pn8_ref[...]
    ri = jax.lax.broadcasted_iota(jnp.int32, (8, 8 * _OUT), 0)
    ci = jax.lax.broadcasted_iota(jnp.int32, (8, 8 * _OUT), 1)
    r8a = (ri == ci // _OUT).astype(jnp.float32)
    for b in range(_GB):
        a2 = xt_ref[b]
        af2 = a2.astype(jnp.float32)
        p1g = jnp.dot(af2, gw8, preferred_element_type=jnp.float32)
        v2 = af2 * (p1g + 2.0 * qw8)
        gqc = jnp.dot(v2, s16, preferred_element_type=jnp.float32)
        mu2 = jnp.dot(af2, w8bar, preferred_element_type=jnp.float32) + pb8
        msq = (gqc + pn8) * (1.0 / _D)
        rs2 = jax.lax.rsqrt(msq - mu2 * mu2 + 1e-6)
        raw2 = jnp.dot(a2, mball8_ref[b],
                       preferred_element_type=jnp.float32) + posw8_ref[b]
        rsb = jnp.dot(rs2, r8a, preferred_element_type=jnp.float32)
        rsmub = jnp.dot(rs2 * mu2, r8a, preferred_element_type=jnp.float32)
        out_ref[b] = (rsb * raw2 - rsmub * soff8_ref[b, 0:1]
                      + soff8_ref[b, 1:2])


def kernel(x, t, y, W_patch, b_patch, pos_embed, freqs, W_t1, b_t1, W_t2, b_t2,
           y_table, W_ada, b_ada, W_proj, b_proj):
    Bb = x.shape[0]
    xb = x.astype(jnp.bfloat16).reshape(Bb, 4, 64, 32, 2)
    xu = jax.lax.bitcast_convert_type(xb, np.uint32)
    xu = xu.reshape(Bb, 4, 32, 2, 32).transpose(0, 2, 4, 1, 3)
    xt = jax.lax.bitcast_convert_type(xu, jnp.bfloat16)
    xt2 = xt.reshape(Bb, _NP, 8 * _K)

    t2 = t.reshape(_B, 1)
    fr2 = freqs.reshape(1, _D // 2)
    y2 = y.reshape(_B, 1).astype(jnp.int32)
    pos2 = pos_embed.reshape(_N, _D)
    wpt = W_patch.T

    s = pl.pallas_call(
        _cond_kernel,
        out_shape=jax.ShapeDtypeStruct((_B, _D), jnp.float32),
    )(t2, fr2, W_t1, b_t1.reshape(1, _D), W_t2, b_t2.reshape(1, _D),
      y2, y_table)

    qw_u, pb_u, pn_u, gw8, s16, w8bar, mball8, posw_u, soff8 = pl.pallas_call(
        _pre_kernel,
        out_shape=(jax.ShapeDtypeStruct((_N, _K), jnp.float32),
                   jax.ShapeDtypeStruct((_N, 1), jnp.float32),
                   jax.ShapeDtypeStruct((_N, 1), jnp.float32),
                   jax.ShapeDtypeStruct((8 * _K, 8 * _K), jnp.float32),
                   jax.ShapeDtypeStruct((8 * _K, 8), jnp.float32),
                   jax.ShapeDtypeStruct((8 * _K, 8), jnp.float32),
                   jax.ShapeDtypeStruct((_B, _NP, 8 * _OUT), jnp.bfloat16),
                   jax.ShapeDtypeStruct((_N, _BO), jnp.float32),
                   jax.ShapeDtypeStruct((_B, 2, 8 * _OUT), jnp.float32)),
    )(s, W_ada, b_ada.reshape(1, 2 * _D), pos2, b_patch.reshape(1, _D), wpt,
      W_proj, b_proj.reshape(1, _OUT))
    qw8 = qw_u.reshape(_NP, 8 * _K)
    pb8 = pb_u.reshape(_NP, 8)
    pn8 = pn_u.reshape(_NP, 8)
    posw8 = posw_u.reshape(_NP, 8, _B, _OUT).transpose(2, 0, 1, 3)
    posw8 = posw8.reshape(_B, _NP, 8 * _OUT)

    out8 = pl.pallas_call(
        _main_kernel,
        grid=(_B // _GB,),
        in_specs=[
            pl.BlockSpec((_GB, _NP, 8 * _K), lambda g: (g, 0, 0)),
            pl.BlockSpec((8 * _K, 8 * _K), lambda g: (0, 0)),
            pl.BlockSpec((8 * _K, 8), lambda g: (0, 0)),
            pl.BlockSpec((8 * _K, 8), lambda g: (0, 0)),
            pl.BlockSpec((_NP, 8 * _K), lambda g: (0, 0)),
            pl.BlockSpec((_NP, 8), lambda g: (0, 0)),
            pl.BlockSpec((_NP, 8), lambda g: (0, 0)),
            pl.BlockSpec((_GB, _NP, 8 * _OUT), lambda g: (g, 0, 0)),
            pl.BlockSpec((_GB, _NP, 8 * _OUT), lambda g: (g, 0, 0)),
            pl.BlockSpec((_GB, 2, 8 * _OUT), lambda g: (g, 0, 0)),
        ],
        out_specs=pl.BlockSpec((_GB, _NP, 8 * _OUT), lambda g: (g, 0, 0)),
        out_shape=jax.ShapeDtypeStruct((_B, _NP, 8 * _OUT), jnp.float32),
    )(xt2, gw8, s16, w8bar, qw8, pb8, pn8, mball8, posw8, soff8)
    return out8.reshape(_B, _N, _OUT)


# speedup vs baseline: 2.1763x; 2.1763x over previous
import jax
import jax.numpy as jnp
import numpy as np
from jax.experimental import pallas as pl

_B = 16
_N = 1024
_D = 1152
_K = 16
_OUT = 32
_BK = _B * _K
_BO = _B * _OUT


def _silu(v):
    return v * jax.nn.sigmoid(v)


def _cond_kernel(t_ref, fr_ref, wt1_ref, bt1_ref, wt2_ref, bt2_ref,
                 y_ref, ytab_ref, s_ref):
    args = t_ref[...] * fr_ref[...]
    emb = jnp.concatenate([jnp.sin(args), jnp.cos(args)], axis=-1)
    h = jnp.dot(emb, wt1_ref[...], preferred_element_type=jnp.float32)
    h = _silu(h + bt1_ref[...])
    temb = jnp.dot(h, wt2_ref[...], preferred_element_type=jnp.float32)
    temb = temb + bt2_ref[...]
    n_cls = ytab_ref.shape[0]
    iota = jax.lax.broadcasted_iota(jnp.int32, (_B, n_cls), 1)
    onehot = (iota == y_ref[...]).astype(jnp.float32)
    yemb = jnp.dot(onehot, ytab_ref[...], preferred_element_type=jnp.float32)
    s_ref[...] = _silu(temb + yemb)


def _pre_kernel(s_ref, wada_ref, bada_ref, pos_ref, bp_ref, wpt_ref,
                wproj_ref, bproj_ref,
                qw_ref, pb_ref, pn_ref, gbig_ref, sbig_ref, wbarbig_ref,
                mbig_ref, posw_ref, soff_ref):
    ada = jnp.dot(s_ref[...], wada_ref[...], preferred_element_type=jnp.float32)
    ada = ada + bada_ref[...]
    shift = ada[:, :_D]
    sc1 = 1.0 + ada[:, _D:]

    posq = pos_ref[...] + bp_ref[...]
    pb_ref[...] = jnp.mean(posq, axis=1, keepdims=True)
    pn_ref[...] = jnp.sum(posq * posq, axis=1, keepdims=True)
    qw_ref[...] = jnp.dot(posq, wpt_ref[...], preferred_element_type=jnp.float32)

    wpt = wpt_ref[...]
    g = jax.lax.dot_general(wpt, wpt, (((0,), (0,)), ((), ())),
                            preferred_element_type=jnp.float32)
    wbarc = jax.lax.dot_general(
        wpt, jnp.full((_D, 1), 1.0 / _D, jnp.float32),
        (((0,), (0,)), ((), ())), preferred_element_type=jnp.float32)

    bi = jax.lax.broadcasted_iota(jnp.int32, (_B, _BO), 0)
    ci = jax.lax.broadcasted_iota(jnp.int32, (_B, _BO), 1)
    rsel = (bi == ci // _OUT).astype(jnp.float32)
    oi = jax.lax.broadcasted_iota(jnp.int32, (_OUT, _BO), 0)
    cj = jax.lax.broadcasted_iota(jnp.int32, (_OUT, _BO), 1)
    tsel = (oi == cj % _OUT).astype(jnp.float32)
    scale_exp = jax.lax.dot_general(sc1, rsel, (((0,), (0,)), ((), ())),
                                    preferred_element_type=jnp.float32)
    wtile = jnp.dot(wproj_ref[...], tsel, preferred_element_type=jnp.float32)
    wball = (scale_exp * wtile).astype(jnp.bfloat16)

    posw_ref[...] = jnp.dot(posq.astype(jnp.bfloat16), wball,
                            preferred_element_type=jnp.float32)
    mball = jax.lax.dot_general(
        wpt.astype(jnp.bfloat16), wball, (((0,), (0,)), ((), ())),
        preferred_element_type=jnp.float32)

    rg = jax.lax.broadcasted_iota(jnp.int32, (_BK, _BK), 0)
    cg = jax.lax.broadcasted_iota(jnp.int32, (_BK, _BK), 1)
    gbig_ref[...] = jnp.tile(g, (_B, _B)) * (rg // _K == cg // _K
                                             ).astype(jnp.float32)
    rs_ = jax.lax.broadcasted_iota(jnp.int32, (_BK, _B), 0)
    cs_ = jax.lax.broadcasted_iota(jnp.int32, (_BK, _B), 1)
    dsel = (rs_ // _K == cs_).astype(jnp.float32)
    sbig_ref[...] = dsel
    wbarbig_ref[...] = dsel * jnp.tile(wbarc, (_B, 1))
    rm = jax.lax.broadcasted_iota(jnp.int32, (_BK, _BO), 0)
    cm = jax.lax.broadcasted_iota(jnp.int32, (_BK, _BO), 1)
    mbig_ref[...] = (jnp.tile(mball, (_B, 1))
                     * (rm // _K == cm // _OUT).astype(jnp.float32)
                     ).astype(jnp.bfloat16)

    s_all = jnp.dot(sc1, wproj_ref[...], preferred_element_type=jnp.float32)
    off_all = jnp.dot(shift, wproj_ref[...],
                      preferred_element_type=jnp.float32) + bproj_ref[...]
    soff_ref[...] = jnp.concatenate(
        [s_all.reshape(1, _B, _OUT), off_all.reshape(1, _B, _OUT)], axis=0)


def _main_kernel(xt_ref, gbig_ref, sbig_ref, wbarbig_ref, qw_ref, pb_ref,
                 pn_ref, mbig_ref, posw_ref, soff_ref, out_ref):
    a2 = xt_ref[...]
    af2 = a2.astype(jnp.float32)
    p1g = jnp.dot(af2, gbig_ref[...], preferred_element_type=jnp.float32)
    qwt = jnp.tile(qw_ref[...], (1, _B))
    v2 = af2 * (p1g + 2.0 * qwt)
    gqc = jnp.dot(v2, sbig_ref[...], preferred_element_type=jnp.float32)
    mu2 = jnp.dot(af2, wbarbig_ref[...],
                  preferred_element_type=jnp.float32) + pb_ref[...]
    msq = (gqc + pn_ref[...]) * (1.0 / _D)
    rs2 = jax.lax.rsqrt(msq - mu2 * mu2 + 1e-6)
    raw = jnp.dot(a2, mbig_ref[...],
                  preferred_element_type=jnp.float32) + posw_ref[...]
    ri = jax.lax.broadcasted_iota(jnp.int32, (_B, _BO), 0)
    ci = jax.lax.broadcasted_iota(jnp.int32, (_B, _BO), 1)
    rb = (ri == ci // _OUT).astype(jnp.float32)
    rsb = jnp.dot(rs2, rb, preferred_element_type=jnp.float32)
    rsmub = jnp.dot(rs2 * mu2, rb, preferred_element_type=jnp.float32)
    out_ref[...] = (rsb * raw - rsmub * soff_ref[0] + soff_ref[1])


def kernel(x, t, y, W_patch, b_patch, pos_embed, freqs, W_t1, b_t1, W_t2, b_t2,
           y_table, W_ada, b_ada, W_proj, b_proj):
    Bb = x.shape[0]
    xb = x.astype(jnp.bfloat16).reshape(Bb, 4, 64, 32, 2)
    xu = jax.lax.bitcast_convert_type(xb, np.uint32)
    xu = xu.reshape(Bb, 4, 32, 2, 32).transpose(2, 4, 0, 1, 3)
    xt = jax.lax.bitcast_convert_type(xu, jnp.bfloat16)
    xt2 = xt.reshape(_N, _BK)

    t2 = t.reshape(_B, 1)
    fr2 = freqs.reshape(1, _D // 2)
    y2 = y.reshape(_B, 1).astype(jnp.int32)
    pos2 = pos_embed.reshape(_N, _D)
    wpt = W_patch.T

    s = pl.pallas_call(
        _cond_kernel,
        out_shape=jax.ShapeDtypeStruct((_B, _D), jnp.float32),
    )(t2, fr2, W_t1, b_t1.reshape(1, _D), W_t2, b_t2.reshape(1, _D),
      y2, y_table)

    (qw, pb, pn, gbig, sbig, wbarbig, mbig, posw, soff_u) = pl.pallas_call(
        _pre_kernel,
        out_shape=(jax.ShapeDtypeStruct((_N, _K), jnp.float32),
                   jax.ShapeDtypeStruct((_N, 1), jnp.float32),
                   jax.ShapeDtypeStruct((_N, 1), jnp.float32),
                   jax.ShapeDtypeStruct((_BK, _BK), jnp.float32),
                   jax.ShapeDtypeStruct((_BK, _B), jnp.float32),
                   jax.ShapeDtypeStruct((_BK, _B), jnp.float32),
                   jax.ShapeDtypeStruct((_BK, _BO), jnp.bfloat16),
                   jax.ShapeDtypeStruct((_N, _BO), jnp.float32),
                   jax.ShapeDtypeStruct((2, _B, _OUT), jnp.float32)),
    )(s, W_ada, b_ada.reshape(1, 2 * _D), pos2, b_patch.reshape(1, _D), wpt,
      W_proj, b_proj.reshape(1, _OUT))
    soff = soff_u.reshape(2, 1, _BO)

    out_all = pl.pallas_call(
        _main_kernel,
        out_shape=jax.ShapeDtypeStruct((_N, _BO), jnp.float32),
    )(xt2, gbig, sbig, wbarbig, qw, pb, pn, mbig, posw, soff)
    out = out_all.reshape(_N, _B, _OUT).transpose(1, 0, 2)
    return out
